# Initial kernel scaffold; baseline (speedup 1.0000x reference)
import jax
import jax.numpy as jnp
from jax import lax
from jax.experimental import pallas as pl
from jax.experimental.pallas import tpu as pltpu
from jax.experimental.pallas import tpu_sc as plsc

_N = 100000
_E = 1600000
_HALF = 16
_NT = 16
_EPT = _E // _NT
_CH = 2000
_NCH = _EPT // _CH
_RPT = _N // _NT
_ECH = 1250
_NECH = _RPT // _ECH


def _sc_body(w2, e1, e2, pos_o, neg_o, h_o, acc, src_v, dst_v, rows_v,
             buf_v, zero_v):
    c = lax.axis_index("c")
    s = lax.axis_index("s")
    row0 = s * _RPT

    def _zset(i, _):
        zero_v[i, :] = jnp.zeros((_HALF,), jnp.float32)
        return 0
    lax.fori_loop(0, _ECH, _zset, 0)

    def _zacc(k, _):
        r0 = pl.multiple_of(row0 + k * _ECH, 8)
        pltpu.sync_copy(zero_v, acc.at[pl.ds(r0, _ECH)])
        return 0
    lax.fori_loop(0, _NECH, _zacc, 0)
    plsc.subcore_barrier()

    def _accumulate(edges, table):
        base = s * _EPT

        def _step(k, _):
            off = pl.multiple_of(base + k * _CH, 8)
            pltpu.sync_copy(edges.at[0, pl.ds(off, _CH)], src_v)
            pltpu.sync_copy(edges.at[1, pl.ds(off, _CH)], dst_v)
            pltpu.sync_copy(table.at[src_v], rows_v)
            pltpu.sync_copy(rows_v, acc.at[dst_v], add=True)
            return 0
        lax.fori_loop(0, _NCH, _step, 0)
        plsc.subcore_barrier()

    def _evacuate(out, do_relu):
        def _step(k, _):
            r0 = pl.multiple_of(row0 + k * _ECH, 8)
            pltpu.sync_copy(acc.at[pl.ds(r0, _ECH)], buf_v)
            if do_relu:
                def _relu_row(i, _):
                    buf_v[i, :] = jnp.maximum(buf_v[i, :], 0.0)
                    return 0
                lax.fori_loop(0, _ECH, _relu_row, 0)
            pltpu.sync_copy(buf_v, out.at[pl.ds(r0, _ECH)])
            pltpu.sync_copy(zero_v, acc.at[pl.ds(r0, _ECH)])
            return 0
        lax.fori_loop(0, _NECH, _step, 0)
        plsc.subcore_barrier()

    for edges, out in ((e1, pos_o), (e2, neg_o)):
        _accumulate(edges, w2.at[c])
        _evacuate(h_o.at[c], True)
        _accumulate(edges, h_o.at[c])
        _evacuate(out.at[c], False)


def kernel(inputs, edge_index1, edge_index2, w):
    del inputs
    w2 = jnp.stack([w[:, :_HALF], w[:, _HALF:]])
    mesh = plsc.VectorSubcoreMesh(core_axis_name="c", subcore_axis_name="s")
    f = pl.kernel(
        _sc_body,
        out_type=[
            jax.ShapeDtypeStruct((2, _N, _HALF), jnp.float32),
            jax.ShapeDtypeStruct((2, _N, _HALF), jnp.float32),
            jax.ShapeDtypeStruct((2, _N, _HALF), jnp.float32),
        ],
        mesh=mesh,
        scratch_types=[
            pltpu.VMEM_SHARED((_N, _HALF), jnp.float32),
            pltpu.VMEM((_CH,), jnp.int32),
            pltpu.VMEM((_CH,), jnp.int32),
            pltpu.VMEM((_CH, _HALF), jnp.float32),
            pltpu.VMEM((_ECH, _HALF), jnp.float32),
            pltpu.VMEM((_ECH, _HALF), jnp.float32),
        ],
    )
    pos2, neg2, _ = f(w2, edge_index1, edge_index2)
    pos = jnp.concatenate([pos2[0], pos2[1]], axis=1)
    neg = jnp.concatenate([neg2[0], neg2[1]], axis=1)
    return pos, neg


# speedup vs baseline: 9.4651x; 9.4651x over previous
import jax
import jax.numpy as jnp
from jax import lax
from jax.experimental import pallas as pl
from jax.experimental.pallas import tpu as pltpu
from jax.experimental.pallas import tpu_sc as plsc

_N = 100000
_E = 1600000
_HALF = 16
_NT = 16
_EPT = _E // _NT
_CH = 1000
_NCH = _EPT // _CH
_RPT = _N // _NT
_ECH = 250
_NECH = _RPT // _ECH


def _sc_body(w2, src1, dst1, src2, dst2, pos_o, neg_o, h_o, acc, src_v,
             dst_v, rows_v, buf_v, zero_v):
    c = lax.axis_index("c")
    s = lax.axis_index("s")
    row0 = s * _RPT

    def _zset(i, _):
        zero_v[i, :] = jnp.zeros((_HALF,), jnp.float32)
        return 0
    lax.fori_loop(0, _ECH, _zset, 0)

    def _zacc(k, _):
        r0 = pl.multiple_of(row0 + k * _ECH, 8)
        pltpu.sync_copy(zero_v, acc.at[pl.ds(r0, _ECH)])
        return 0
    lax.fori_loop(0, _NECH, _zacc, 0)
    plsc.subcore_barrier()

    def _accumulate(src_e, dst_e, table):
        base = s * _EPT

        def _step(k, _):
            off = pl.multiple_of(base + k * _CH, 8)
            pltpu.sync_copy(src_e.at[pl.ds(off, _CH)], src_v)
            pltpu.sync_copy(dst_e.at[pl.ds(off, _CH)], dst_v)
            pltpu.sync_copy(table.at[src_v], rows_v)
            pltpu.sync_copy(rows_v, acc.at[dst_v], add=True)
            return 0
        lax.fori_loop(0, _NCH, _step, 0)
        plsc.subcore_barrier()

    def _evacuate(out, do_relu):
        def _step(k, _):
            r0 = pl.multiple_of(row0 + k * _ECH, 8)
            pltpu.sync_copy(acc.at[pl.ds(r0, _ECH)], buf_v)
            if do_relu:
                def _relu_row(i, _):
                    buf_v[i, :] = jnp.maximum(buf_v[i, :], 0.0)
                    return 0
                lax.fori_loop(0, _ECH, _relu_row, 0)
            pltpu.sync_copy(buf_v, out.at[pl.ds(r0, _ECH)])
            pltpu.sync_copy(zero_v, acc.at[pl.ds(r0, _ECH)])
            return 0
        lax.fori_loop(0, _NECH, _step, 0)
        plsc.subcore_barrier()

    for src_e, dst_e, out in ((src1, dst1, pos_o), (src2, dst2, neg_o)):
        _accumulate(src_e, dst_e, w2.at[c])
        _evacuate(h_o.at[c], True)
        _accumulate(src_e, dst_e, h_o.at[c])
        _evacuate(out.at[c], False)


def kernel(inputs, edge_index1, edge_index2, w):
    del inputs
    w2 = jnp.stack([w[:, :_HALF], w[:, _HALF:]])
    mesh = plsc.VectorSubcoreMesh(core_axis_name="c", subcore_axis_name="s")
    f = pl.kernel(
        _sc_body,
        out_type=[
            jax.ShapeDtypeStruct((2, _N, _HALF), jnp.float32),
            jax.ShapeDtypeStruct((2, _N, _HALF), jnp.float32),
            jax.ShapeDtypeStruct((2, _N, _HALF), jnp.float32),
        ],
        mesh=mesh,
        scratch_types=[
            pltpu.VMEM_SHARED((_N, _HALF), jnp.float32),
            pltpu.VMEM((_CH,), jnp.int32),
            pltpu.VMEM((_CH,), jnp.int32),
            pltpu.VMEM((_CH, _HALF), jnp.float32),
            pltpu.VMEM((_ECH, _HALF), jnp.float32),
            pltpu.VMEM((_ECH, _HALF), jnp.float32),
        ],
        compiler_params=pltpu.CompilerParams(use_tc_tiling_on_sc=False),
    )
    pos2, neg2, _ = f(w2, edge_index1[0], edge_index1[1],
                      edge_index2[0], edge_index2[1])
    pos = jnp.concatenate([pos2[0], pos2[1]], axis=1)
    neg = jnp.concatenate([neg2[0], neg2[1]], axis=1)
    return pos, neg
